# Initial kernel scaffold; baseline (speedup 1.0000x reference)
import jax
import jax.numpy as jnp
from jax.experimental import pallas as pl


def kernel(tenInput, tenFlow, tenMetric):
    raise NotImplementedError("write your pallas kernel here")


# speedup vs baseline: 1.5912x; 1.5912x over previous
import functools

import jax
import jax.numpy as jnp
from jax import lax
from jax.experimental import pallas as pl
from jax.experimental.pallas import tpu as pltpu
from jax.experimental.pallas import tpu_sc as plsc

B, C, H, W = 2, 96, 512, 512
HW = H * W
NC, NS, L = 2, 16, 16
SLICE = HW // NS
WIN = 2048
NWIN = SLICE // WIN
G = 4
NG = C // G
ROWS_PER_BLK = 64


def _precompute_body(flow_ref, metric_ref, idx_ref, w_ref):
    r = pl.program_id(1) * ROWS_PER_BLK
    u = flow_ref[0, 0]
    v = flow_ref[0, 1]
    e = jnp.exp(metric_ref[0, 0])
    xi = lax.broadcasted_iota(jnp.int32, (ROWS_PER_BLK, W), 1)
    yi = lax.broadcasted_iota(jnp.int32, (ROWS_PER_BLK, W), 0) + r
    x = xi.astype(jnp.float32)
    y = yi.astype(jnp.float32)
    flt_x = x + u
    flt_y = y + v
    ix0f = jnp.floor(flt_x)
    iy0f = jnp.floor(flt_y)
    fx = flt_x - ix0f
    fy = flt_y - iy0f
    ix0 = ix0f.astype(jnp.int32)
    iy0 = iy0f.astype(jnp.int32)
    p_self = yi * W + xi
    for k, (ix, iy, wk) in enumerate((
            (ix0, iy0, (1.0 - fx) * (1.0 - fy)),
            (ix0 + 1, iy0, fx * (1.0 - fy)),
            (ix0, iy0 + 1, (1.0 - fx) * fy),
            (ix0 + 1, iy0 + 1, fx * fy),
    )):
        valid = (ix >= 0) & (ix < W) & (iy >= 0) & (iy < H)
        idx_ref[0, k] = jnp.where(valid, iy * W + ix, p_self)
        w_ref[0, k] = jnp.where(valid, wk * e, 0.0)


def _precompute(tenFlow, tenMetric):
    grid = (B, H // ROWS_PER_BLK)
    return pl.pallas_call(
        _precompute_body,
        grid=grid,
        in_specs=[
            pl.BlockSpec((1, 2, ROWS_PER_BLK, W), lambda b, i: (b, 0, i, 0)),
            pl.BlockSpec((1, 1, ROWS_PER_BLK, W), lambda b, i: (b, 0, i, 0)),
        ],
        out_specs=[
            pl.BlockSpec((1, 4, ROWS_PER_BLK, W), lambda b, i: (b, 0, i, 0)),
            pl.BlockSpec((1, 4, ROWS_PER_BLK, W), lambda b, i: (b, 0, i, 0)),
        ],
        out_shape=[
            jax.ShapeDtypeStruct((B, 4, H, W), jnp.int32),
            jax.ShapeDtypeStruct((B, 4, H, W), jnp.float32),
        ],
    )(tenFlow, tenMetric)


_SC_SCRATCH = dict(
    val_v=pltpu.VMEM((G, WIN), jnp.float32),
    upd_v=pltpu.VMEM((WIN,), jnp.float32),
    zbuf=pltpu.VMEM((WIN,), jnp.float32),
    rnorm_v=pltpu.VMEM((SLICE,), jnp.float32),
    outbuf=pltpu.VMEM((WIN,), jnp.float32),
)
_SC_SCRATCH.update({f"idx{k}": pltpu.VMEM((WIN,), jnp.int32) for k in range(4)})
_SC_SCRATCH.update({f"wgt{k}": pltpu.VMEM((WIN,), jnp.float32) for k in range(4)})
_SC_SCRATCH.update({f"plane{g}": pltpu.VMEM_SHARED((HW,), jnp.float32)
                    for g in range(G)})


@functools.partial(
    pl.kernel,
    out_type=jax.ShapeDtypeStruct((B, C, HW), jnp.float32),
    mesh=plsc.VectorSubcoreMesh(core_axis_name="c", subcore_axis_name="s",
                                num_cores=NC, num_subcores=NS),
    scratch_types=_SC_SCRATCH,
)
def _splat(in_hbm, idx_hbm, w_hbm, out_hbm, val_v,
           upd_v, zbuf, rnorm_v, outbuf,
           idx0, idx1, idx2, idx3, wgt0, wgt1, wgt2, wgt3,
           plane0, plane1, plane2, plane3):
    planes = (plane0, plane1, plane2, plane3)
    idxs = (idx0, idx1, idx2, idx3)
    wgts = (wgt0, wgt1, wgt2, wgt3)
    cid = lax.axis_index("c")
    sid = lax.axis_index("s")
    my = pl.ds(sid * SLICE, SLICE)

    def zfill(i):
        zbuf[pl.ds(i * L, L)] = jnp.zeros((L,), jnp.float32)
    pl.loop(0, WIN // L)(zfill)

    def zero_slice(plane):
        def z(j):
            pltpu.sync_copy(zbuf, plane.at[pl.ds(sid * SLICE + j * WIN, WIN)])
        pl.loop(0, NWIN)(z)

    def stage_idx_w(w):
        base = sid * SLICE + w * WIN
        for k in range(4):
            pltpu.sync_copy(idx_hbm.at[cid, k, pl.ds(base, WIN)], idxs[k])
            pltpu.sync_copy(w_hbm.at[cid, k, pl.ds(base, WIN)], wgts[k])

    zero_slice(plane0)
    plsc.subcore_barrier()

    def norm_win(w):
        stage_idx_w(w)
        for k in range(4):
            pltpu.sync_copy(wgts[k], plane0.at[idxs[k]], add=True)
    pl.loop(0, NWIN)(norm_win)
    plsc.subcore_barrier()

    pltpu.sync_copy(plane0.at[my], rnorm_v)

    def recip(i):
        s = pl.ds(i * L, L)
        nv = rnorm_v[s]
        rnorm_v[s] = 1.0 / jnp.where(nv == 0.0, 1.0, nv)
    pl.loop(0, SLICE // L)(recip)

    def group(g):
        for gg in range(G):
            zero_slice(planes[gg])
        plsc.subcore_barrier()

        def win(w):
            base = sid * SLICE + w * WIN
            stage_idx_w(w)
            for gg in range(G):
                pltpu.sync_copy(in_hbm.at[cid, g * G + gg, pl.ds(base, WIN)],
                                val_v.at[gg])
            for gg in range(G):
                for k in range(4):
                    def mul(i, gg=gg, k=k):
                        s = pl.ds(i * L, L)
                        upd_v[s] = wgts[k][s] * val_v[gg, s]
                    pl.loop(0, WIN // L)(mul)
                    pltpu.sync_copy(upd_v, planes[gg].at[idxs[k]], add=True)
        pl.loop(0, NWIN)(win)
        plsc.subcore_barrier()

        for gg in range(G):
            def flush(j, gg=gg):
                base = sid * SLICE + j * WIN
                pltpu.sync_copy(planes[gg].at[pl.ds(base, WIN)], outbuf)

                def norm_mul(i):
                    s = pl.ds(i * L, L)
                    outbuf[s] = outbuf[s] * rnorm_v[pl.ds(j * WIN + i * L, L)]
                pl.loop(0, WIN // L)(norm_mul)
                pltpu.sync_copy(outbuf,
                                out_hbm.at[cid, g * G + gg, pl.ds(base, WIN)])
            pl.loop(0, NWIN)(flush)
    pl.loop(0, NG)(group)


def kernel(tenInput, tenFlow, tenMetric):
    idx4, w4 = _precompute(tenFlow, tenMetric)
    out = _splat(tenInput.reshape(B, C, HW),
                 idx4.reshape(B, 4, HW),
                 w4.reshape(B, 4, HW))
    return out.reshape(B, C, H, W)


# speedup vs baseline: 2.8559x; 1.7948x over previous
import functools

import jax
import jax.numpy as jnp
from jax import lax
from jax.experimental import pallas as pl
from jax.experimental.pallas import tpu as pltpu
from jax.experimental.pallas import tpu_sc as plsc

B, C, H, W = 2, 96, 512, 512
HW = H * W
NC, NS, L = 2, 16, 16
SLICE = HW // NS
WIN = 1024
NWIN = SLICE // WIN
G = 4
NGRP = C // G
ZCHUNK = 2048
ROWS_PER_BLK = 64


def _precompute_body(flow_ref, metric_ref, idx_ref, w_ref):
    r = pl.program_id(1) * ROWS_PER_BLK
    u = flow_ref[0, 0]
    v = flow_ref[0, 1]
    e = jnp.exp(metric_ref[0, 0])
    xi = lax.broadcasted_iota(jnp.int32, (ROWS_PER_BLK, W), 1)
    yi = lax.broadcasted_iota(jnp.int32, (ROWS_PER_BLK, W), 0) + r
    x = xi.astype(jnp.float32)
    y = yi.astype(jnp.float32)
    flt_x = x + u
    flt_y = y + v
    ix0f = jnp.floor(flt_x)
    iy0f = jnp.floor(flt_y)
    fx = flt_x - ix0f
    fy = flt_y - iy0f
    ix0 = ix0f.astype(jnp.int32)
    iy0 = iy0f.astype(jnp.int32)
    p_self = yi * W + xi
    for k, (ix, iy, wk) in enumerate((
            (ix0, iy0, (1.0 - fx) * (1.0 - fy)),
            (ix0 + 1, iy0, fx * (1.0 - fy)),
            (ix0, iy0 + 1, (1.0 - fx) * fy),
            (ix0 + 1, iy0 + 1, fx * fy),
    )):
        valid = (ix >= 0) & (ix < W) & (iy >= 0) & (iy < H)
        idx_ref[0, k] = jnp.where(valid, iy * W + ix, p_self)
        w_ref[0, k] = jnp.where(valid, wk * e, 0.0)


def _precompute(tenFlow, tenMetric):
    grid = (B, H // ROWS_PER_BLK)
    return pl.pallas_call(
        _precompute_body,
        grid=grid,
        in_specs=[
            pl.BlockSpec((1, 2, ROWS_PER_BLK, W), lambda b, i: (b, 0, i, 0)),
            pl.BlockSpec((1, 1, ROWS_PER_BLK, W), lambda b, i: (b, 0, i, 0)),
        ],
        out_specs=[
            pl.BlockSpec((1, 4, ROWS_PER_BLK, W), lambda b, i: (b, 0, i, 0)),
            pl.BlockSpec((1, 4, ROWS_PER_BLK, W), lambda b, i: (b, 0, i, 0)),
        ],
        out_shape=[
            jax.ShapeDtypeStruct((B, 4, H, W), jnp.int32),
            jax.ShapeDtypeStruct((B, 4, H, W), jnp.float32),
        ],
    )(tenFlow, tenMetric)


_SC_SCRATCH = dict(
    val_v=pltpu.VMEM((G * WIN,), jnp.float32),
    zbuf=pltpu.VMEM((ZCHUNK,), jnp.float32),
    fbuf=pltpu.VMEM((WIN,), jnp.float32),
    rnorm_v=pltpu.VMEM((SLICE,), jnp.float32),
    sem_in=pltpu.SemaphoreType.DMA,
    sem_sc=pltpu.SemaphoreType.DMA,
    sem_z=pltpu.SemaphoreType.DMA,
    sem_out=pltpu.SemaphoreType.DMA,
)
_SC_SCRATCH.update({f"idx{k}": pltpu.VMEM((WIN,), jnp.int32) for k in range(4)})
_SC_SCRATCH.update({f"wgt{k}": pltpu.VMEM((WIN,), jnp.float32) for k in range(4)})
_SC_SCRATCH.update({f"upd{k}_{c}": pltpu.VMEM((WIN,), jnp.float32)
                    for k in range(4) for c in range(G)})
_SC_SCRATCH.update({f"plane{g}": pltpu.VMEM_SHARED((HW,), jnp.float32)
                    for g in range(G)})
_SC_SCRATCH.update({f"obuf{c}": pltpu.VMEM((WIN,), jnp.float32)
                    for c in range(G)})


@functools.partial(
    pl.kernel,
    out_type=jax.ShapeDtypeStruct((B, C, HW), jnp.float32),
    mesh=plsc.VectorSubcoreMesh(core_axis_name="c", subcore_axis_name="s",
                                num_cores=NC, num_subcores=NS),
    scratch_types=_SC_SCRATCH,
)
def _splat(in_hbm, idx_hbm, w_hbm, out_hbm, val_v, zbuf, fbuf, rnorm_v,
           sem_in, sem_sc, sem_z, sem_out, **refs):
    idxs = tuple(refs[f"idx{k}"] for k in range(4))
    wgts = tuple(refs[f"wgt{k}"] for k in range(4))
    upds = tuple(tuple(refs[f"upd{k}_{c}"] for c in range(G)) for k in range(4))
    planes = tuple(refs[f"plane{g}"] for g in range(G))
    obufs = tuple(refs[f"obuf{c}"] for c in range(G))
    cid = lax.axis_index("c")
    sid = lax.axis_index("s")

    def zfill(i):
        zbuf[pl.ds(i * L, L)] = jnp.zeros((L,), jnp.float32)
    pl.loop(0, ZCHUNK // L)(zfill)

    def zero_planes(ps):
        def zf(j):
            for p in ps:
                pltpu.async_copy(
                    zbuf, p.at[pl.ds(sid * SLICE + j * ZCHUNK, ZCHUNK)], sem_z)
        pl.loop(0, SLICE // ZCHUNK)(zf)

        def zw(j):
            for p in ps:
                pltpu.make_async_copy(
                    zbuf, p.at[pl.ds(sid * SLICE + j * ZCHUNK, ZCHUNK)],
                    sem_z).wait()
        pl.loop(0, SLICE // ZCHUNK)(zw)

    def stage(w, nval, g):
        base = sid * SLICE + w * WIN
        for k in range(4):
            pltpu.async_copy(idx_hbm.at[cid, k, pl.ds(base, WIN)], idxs[k],
                             sem_in)
            pltpu.async_copy(w_hbm.at[cid, k, pl.ds(base, WIN)], wgts[k],
                             sem_in)
        for c in range(nval):
            pltpu.async_copy(in_hbm.at[cid, g * G + c, pl.ds(base, WIN)],
                             val_v.at[pl.ds(c * WIN, WIN)], sem_in)
        for k in range(4):
            pltpu.make_async_copy(idx_hbm.at[cid, k, pl.ds(base, WIN)],
                                  idxs[k], sem_in).wait()
            pltpu.make_async_copy(w_hbm.at[cid, k, pl.ds(base, WIN)],
                                  wgts[k], sem_in).wait()
        for c in range(nval):
            pltpu.make_async_copy(in_hbm.at[cid, g * G + c, pl.ds(base, WIN)],
                                  val_v.at[pl.ds(c * WIN, WIN)],
                                  sem_in).wait()

    zero_planes(planes[:1])
    plsc.subcore_barrier()

    def norm_win(w):
        stage(w, 0, 0)
        for k in range(4):
            pltpu.async_copy(wgts[k], planes[0].at[idxs[k]], sem_sc, add=True)
        for k in range(4):
            pltpu.make_async_copy(wgts[k], planes[0].at[idxs[k]],
                                  sem_sc).wait()
    pl.loop(0, NWIN)(norm_win)
    plsc.subcore_barrier()

    def rext(j):
        base = sid * SLICE + j * WIN
        pltpu.sync_copy(planes[0].at[pl.ds(base, WIN)], fbuf)

        def rb(i):
            v16 = fbuf[pl.ds(i * L, L)]
            rnorm_v[pl.ds(j * WIN + i * L, L)] = 1.0 / jnp.where(
                v16 == 0.0, 1.0, v16)
        pl.loop(0, WIN // L)(rb)
    pl.loop(0, NWIN)(rext)

    def group(g):
        zero_planes(planes)
        plsc.subcore_barrier()

        def win(w):
            stage(w, G, g)
            for k in range(4):
                for c in range(G):
                    def build(i, k=k, c=c):
                        s = pl.ds(i * L, L)
                        upds[k][c][s] = wgts[k][s] * val_v[
                            pl.ds(c * WIN + i * L, L)]
                    pl.loop(0, WIN // L)(build)
                    pltpu.async_copy(upds[k][c], planes[c].at[idxs[k]],
                                     sem_sc, add=True)
            for k in range(4):
                for c in range(G):
                    pltpu.make_async_copy(upds[k][c], planes[c].at[idxs[k]],
                                          sem_sc).wait()
        pl.loop(0, NWIN)(win)
        plsc.subcore_barrier()

        def flush(j):
            base = sid * SLICE + j * WIN
            for c in range(G):
                pltpu.sync_copy(planes[c].at[pl.ds(base, WIN)], obufs[c])

                def fb(i, c=c):
                    s = pl.ds(i * L, L)
                    obufs[c][s] = obufs[c][s] * rnorm_v[pl.ds(j * WIN + i * L,
                                                              L)]
                pl.loop(0, WIN // L)(fb)
                pltpu.async_copy(obufs[c], out_hbm.at[cid, g * G + c,
                                                      pl.ds(base, WIN)],
                                 sem_out)
            for c in range(G):
                pltpu.make_async_copy(obufs[c],
                                      out_hbm.at[cid, g * G + c,
                                                 pl.ds(base, WIN)],
                                      sem_out).wait()
        pl.loop(0, NWIN)(flush)
    pl.loop(0, NGRP)(group)


def kernel(tenInput, tenFlow, tenMetric):
    idx4, w4 = _precompute(tenFlow, tenMetric)
    out = _splat(tenInput.reshape(B, C, HW),
                 idx4.reshape(B, 4, HW),
                 w4.reshape(B, 4, HW))
    return out.reshape(B, C, H, W)


# speedup vs baseline: 2.9091x; 1.0186x over previous
import functools

import jax
import jax.numpy as jnp
from jax import lax
from jax.experimental import pallas as pl
from jax.experimental.pallas import tpu as pltpu
from jax.experimental.pallas import tpu_sc as plsc

B, C, H, W = 2, 96, 512, 512
HW = H * W
NC, NS, L = 2, 16, 16
SLICE = HW // NS
WIN = 1024
NWIN = SLICE // WIN
G = 4
NGRP = C // G
ZCHUNK = 2048
ROWS_PER_BLK = 64


def _precompute_body(flow_ref, metric_ref, idx_ref, w_ref):
    r = pl.program_id(1) * ROWS_PER_BLK
    u = flow_ref[0, 0]
    v = flow_ref[0, 1]
    e = jnp.exp(metric_ref[0, 0])
    xi = lax.broadcasted_iota(jnp.int32, (ROWS_PER_BLK, W), 1)
    yi = lax.broadcasted_iota(jnp.int32, (ROWS_PER_BLK, W), 0) + r
    x = xi.astype(jnp.float32)
    y = yi.astype(jnp.float32)
    flt_x = x + u
    flt_y = y + v
    ix0f = jnp.floor(flt_x)
    iy0f = jnp.floor(flt_y)
    fx = flt_x - ix0f
    fy = flt_y - iy0f
    ix0 = ix0f.astype(jnp.int32)
    iy0 = iy0f.astype(jnp.int32)
    p_self = yi * W + xi
    for k, (ix, iy, wk) in enumerate((
            (ix0, iy0, (1.0 - fx) * (1.0 - fy)),
            (ix0 + 1, iy0, fx * (1.0 - fy)),
            (ix0, iy0 + 1, (1.0 - fx) * fy),
            (ix0 + 1, iy0 + 1, fx * fy),
    )):
        valid = (ix >= 0) & (ix < W) & (iy >= 0) & (iy < H)
        idx_ref[0, k] = jnp.where(valid, iy * W + ix, p_self)
        w_ref[0, k] = jnp.where(valid, wk * e, 0.0)


def _precompute(tenFlow, tenMetric):
    grid = (B, H // ROWS_PER_BLK)
    return pl.pallas_call(
        _precompute_body,
        grid=grid,
        in_specs=[
            pl.BlockSpec((1, 2, ROWS_PER_BLK, W), lambda b, i: (b, 0, i, 0)),
            pl.BlockSpec((1, 1, ROWS_PER_BLK, W), lambda b, i: (b, 0, i, 0)),
        ],
        out_specs=[
            pl.BlockSpec((1, 4, ROWS_PER_BLK, W), lambda b, i: (b, 0, i, 0)),
            pl.BlockSpec((1, 4, ROWS_PER_BLK, W), lambda b, i: (b, 0, i, 0)),
        ],
        out_shape=[
            jax.ShapeDtypeStruct((B, 4, H, W), jnp.int32),
            jax.ShapeDtypeStruct((B, 4, H, W), jnp.float32),
        ],
    )(tenFlow, tenMetric)


_SC_SCRATCH = dict(
    zbuf=pltpu.VMEM((ZCHUNK,), jnp.float32),
    fbuf=pltpu.VMEM((WIN,), jnp.float32),
    rnorm_v=pltpu.VMEM((SLICE,), jnp.float32),
    sem_inA=pltpu.SemaphoreType.DMA,
    sem_inB=pltpu.SemaphoreType.DMA,
    sem_sc=pltpu.SemaphoreType.DMA,
    sem_z=pltpu.SemaphoreType.DMA,
    sem_out=pltpu.SemaphoreType.DMA,
)
for _p in ("A", "B"):
    _SC_SCRATCH[f"val{_p}"] = pltpu.VMEM((G * WIN,), jnp.float32)
    _SC_SCRATCH.update({f"idx{k}{_p}": pltpu.VMEM((WIN,), jnp.int32)
                        for k in range(4)})
    _SC_SCRATCH.update({f"wgt{k}{_p}": pltpu.VMEM((WIN,), jnp.float32)
                        for k in range(4)})
_SC_SCRATCH.update({f"upd{k}_{c}": pltpu.VMEM((WIN,), jnp.float32)
                    for k in range(4) for c in range(G)})
_SC_SCRATCH.update({f"plane{g}": pltpu.VMEM_SHARED((HW,), jnp.float32)
                    for g in range(G)})
_SC_SCRATCH.update({f"obuf{c}": pltpu.VMEM((WIN,), jnp.float32)
                    for c in range(G)})


@functools.partial(
    pl.kernel,
    out_type=jax.ShapeDtypeStruct((B, C, HW), jnp.float32),
    mesh=plsc.VectorSubcoreMesh(core_axis_name="c", subcore_axis_name="s",
                                num_cores=NC, num_subcores=NS),
    scratch_types=_SC_SCRATCH,
)
def _splat(in_hbm, idx_hbm, w_hbm, out_hbm, zbuf, fbuf, rnorm_v,
           sem_inA, sem_inB, sem_sc, sem_z, sem_out, **refs):
    sets = {}
    for p, sem in (("A", sem_inA), ("B", sem_inB)):
        sets[p] = (tuple(refs[f"idx{k}{p}"] for k in range(4)),
                   tuple(refs[f"wgt{k}{p}"] for k in range(4)),
                   refs[f"val{p}"], sem)
    upds = tuple(tuple(refs[f"upd{k}_{c}"] for c in range(G)) for k in range(4))
    planes = tuple(refs[f"plane{g}"] for g in range(G))
    obufs = tuple(refs[f"obuf{c}"] for c in range(G))
    cid = lax.axis_index("c")
    sid = lax.axis_index("s")

    def zfill(i):
        zbuf[pl.ds(i * L, L)] = jnp.zeros((L,), jnp.float32)
    pl.loop(0, ZCHUNK // L)(zfill)

    def zero_planes(ps):
        def zf(j):
            for p in ps:
                pltpu.async_copy(
                    zbuf, p.at[pl.ds(sid * SLICE + j * ZCHUNK, ZCHUNK)], sem_z)
        pl.loop(0, SLICE // ZCHUNK)(zf)

        def zw(j):
            for p in ps:
                pltpu.make_async_copy(
                    zbuf, p.at[pl.ds(sid * SLICE + j * ZCHUNK, ZCHUNK)],
                    sem_z).wait()
        pl.loop(0, SLICE // ZCHUNK)(zw)

    def stage_fire(w, nval, g, p):
        idxs, wgts, val_v, sem = sets[p]
        base = sid * SLICE + w * WIN
        for k in range(4):
            pltpu.async_copy(idx_hbm.at[cid, k, pl.ds(base, WIN)], idxs[k],
                             sem)
            pltpu.async_copy(w_hbm.at[cid, k, pl.ds(base, WIN)], wgts[k],
                             sem)
        for c in range(nval):
            pltpu.async_copy(in_hbm.at[cid, g * G + c, pl.ds(base, WIN)],
                             val_v.at[pl.ds(c * WIN, WIN)], sem)

    def stage_wait(w, nval, g, p):
        idxs, wgts, val_v, sem = sets[p]
        base = sid * SLICE + w * WIN
        for k in range(4):
            pltpu.make_async_copy(idx_hbm.at[cid, k, pl.ds(base, WIN)],
                                  idxs[k], sem).wait()
            pltpu.make_async_copy(w_hbm.at[cid, k, pl.ds(base, WIN)],
                                  wgts[k], sem).wait()
        for c in range(nval):
            pltpu.make_async_copy(in_hbm.at[cid, g * G + c, pl.ds(base, WIN)],
                                  val_v.at[pl.ds(c * WIN, WIN)], sem).wait()

    def paired_windows(nval, g, process):
        stage_fire(0, nval, g, "A")

        def pair(q):
            wa = 2 * q
            wb = 2 * q + 1
            wnext = jnp.minimum(wb + 1, NWIN - 1)
            stage_fire(wb, nval, g, "B")
            stage_wait(wa, nval, g, "A")
            process("A")
            stage_fire(wnext, nval, g, "A")
            stage_wait(wb, nval, g, "B")
            process("B")
        pl.loop(0, NWIN // 2)(pair)
        stage_wait(NWIN - 1, nval, g, "A")

    zero_planes(planes[:1])
    plsc.subcore_barrier()

    def nproc(p):
        idxs, wgts, _, _ = sets[p]
        for k in range(4):
            pltpu.async_copy(wgts[k], planes[0].at[idxs[k]], sem_sc, add=True)
        for k in range(4):
            pltpu.make_async_copy(wgts[k], planes[0].at[idxs[k]],
                                  sem_sc).wait()
    paired_windows(0, 0, nproc)
    plsc.subcore_barrier()

    def rext(j):
        base = sid * SLICE + j * WIN
        pltpu.sync_copy(planes[0].at[pl.ds(base, WIN)], fbuf)

        def rb(i):
            v16 = fbuf[pl.ds(i * L, L)]
            rnorm_v[pl.ds(j * WIN + i * L, L)] = 1.0 / jnp.where(
                v16 == 0.0, 1.0, v16)
        pl.loop(0, WIN // L)(rb)
    pl.loop(0, NWIN)(rext)

    def group(g):
        zero_planes(planes)
        plsc.subcore_barrier()

        def gproc(p):
            idxs, wgts, val_v, _ = sets[p]
            for k in range(4):
                def build(i, k=k):
                    s = pl.ds(i * L, L)
                    w16 = wgts[k][s]
                    for c in range(G):
                        upds[k][c][s] = w16 * val_v[pl.ds(c * WIN + i * L, L)]
                pl.loop(0, WIN // L)(build)
                for c in range(G):
                    pltpu.async_copy(upds[k][c], planes[c].at[idxs[k]],
                                     sem_sc, add=True)
            for k in range(4):
                for c in range(G):
                    pltpu.make_async_copy(upds[k][c], planes[c].at[idxs[k]],
                                          sem_sc).wait()
        paired_windows(G, g, gproc)
        plsc.subcore_barrier()

        def flush(j):
            base = sid * SLICE + j * WIN
            for c in range(G):
                pltpu.sync_copy(planes[c].at[pl.ds(base, WIN)], obufs[c])

                def fb(i, c=c):
                    s = pl.ds(i * L, L)
                    obufs[c][s] = obufs[c][s] * rnorm_v[pl.ds(j * WIN + i * L,
                                                              L)]
                pl.loop(0, WIN // L)(fb)
                pltpu.async_copy(obufs[c], out_hbm.at[cid, g * G + c,
                                                      pl.ds(base, WIN)],
                                 sem_out)
            for c in range(G):
                pltpu.make_async_copy(obufs[c],
                                      out_hbm.at[cid, g * G + c,
                                                 pl.ds(base, WIN)],
                                      sem_out).wait()
        pl.loop(0, NWIN)(flush)
    pl.loop(0, NGRP)(group)


def kernel(tenInput, tenFlow, tenMetric):
    idx4, w4 = _precompute(tenFlow, tenMetric)
    out = _splat(tenInput.reshape(B, C, HW),
                 idx4.reshape(B, 4, HW),
                 w4.reshape(B, 4, HW))
    return out.reshape(B, C, H, W)


# speedup vs baseline: 3.3270x; 1.1437x over previous
import functools

import jax
import jax.numpy as jnp
from jax import lax
from jax.experimental import pallas as pl
from jax.experimental.pallas import tpu as pltpu
from jax.experimental.pallas import tpu_sc as plsc

B, C, H, W = 2, 96, 512, 512
HW = H * W
NC, NS, L = 2, 16, 16
SLICE = HW // NS
WIN = 1024
NWIN = SLICE // WIN
G = 4
NGRP = C // G
ZCHUNK = 2048
ABLATE_G = 0
ROWS_PER_BLK = 64


def _precompute_body(flow_ref, metric_ref, idx_ref, w_ref):
    r = pl.program_id(1) * ROWS_PER_BLK
    u = flow_ref[0, 0]
    v = flow_ref[0, 1]
    e = jnp.exp(metric_ref[0, 0])
    xi = lax.broadcasted_iota(jnp.int32, (ROWS_PER_BLK, W), 1)
    yi = lax.broadcasted_iota(jnp.int32, (ROWS_PER_BLK, W), 0) + r
    x = xi.astype(jnp.float32)
    y = yi.astype(jnp.float32)
    flt_x = x + u
    flt_y = y + v
    ix0f = jnp.floor(flt_x)
    iy0f = jnp.floor(flt_y)
    fx = flt_x - ix0f
    fy = flt_y - iy0f
    ix0 = ix0f.astype(jnp.int32)
    iy0 = iy0f.astype(jnp.int32)
    p_self = yi * W + xi
    for k, (ix, iy, wk) in enumerate((
            (ix0, iy0, (1.0 - fx) * (1.0 - fy)),
            (ix0 + 1, iy0, fx * (1.0 - fy)),
            (ix0, iy0 + 1, (1.0 - fx) * fy),
            (ix0 + 1, iy0 + 1, fx * fy),
    )):
        valid = (ix >= 0) & (ix < W) & (iy >= 0) & (iy < H)
        idx_ref[0, k] = jnp.where(valid, iy * W + ix, p_self)
        w_ref[0, k] = jnp.where(valid, wk * e, 0.0)


def _precompute(tenFlow, tenMetric):
    grid = (B, H // ROWS_PER_BLK)
    return pl.pallas_call(
        _precompute_body,
        grid=grid,
        in_specs=[
            pl.BlockSpec((1, 2, ROWS_PER_BLK, W), lambda b, i: (b, 0, i, 0)),
            pl.BlockSpec((1, 1, ROWS_PER_BLK, W), lambda b, i: (b, 0, i, 0)),
        ],
        out_specs=[
            pl.BlockSpec((1, 4, ROWS_PER_BLK, W), lambda b, i: (b, 0, i, 0)),
            pl.BlockSpec((1, 4, ROWS_PER_BLK, W), lambda b, i: (b, 0, i, 0)),
        ],
        out_shape=[
            jax.ShapeDtypeStruct((B, 4, H, W), jnp.int32),
            jax.ShapeDtypeStruct((B, 4, H, W), jnp.float32),
        ],
    )(tenFlow, tenMetric)


_SC_SCRATCH = dict(
    zbuf=pltpu.VMEM((ZCHUNK,), jnp.float32),
    fbuf=pltpu.VMEM((WIN,), jnp.float32),
    rnorm_v=pltpu.VMEM((SLICE,), jnp.float32),
    sem_inA=pltpu.SemaphoreType.DMA,
    sem_inB=pltpu.SemaphoreType.DMA,
    sem_sc=pltpu.SemaphoreType.DMA,
    sem_z=pltpu.SemaphoreType.DMA,
    sem_out=pltpu.SemaphoreType.DMA,
)
for _p in ("A", "B"):
    _SC_SCRATCH[f"val{_p}"] = pltpu.VMEM((G * WIN,), jnp.float32)
    _SC_SCRATCH.update({f"idx{k}{_p}": pltpu.VMEM((WIN,), jnp.int32)
                        for k in range(4)})
    _SC_SCRATCH.update({f"wgt{k}{_p}": pltpu.VMEM((WIN,), jnp.float32)
                        for k in range(4)})
_SC_SCRATCH.update({f"upd{k}_{c}": pltpu.VMEM((WIN,), jnp.float32)
                    for k in range(4) for c in range(G)})
_SC_SCRATCH.update({f"plane{g}": pltpu.VMEM_SHARED((HW,), jnp.float32)
                    for g in range(G)})
_SC_SCRATCH.update({f"obuf{c}": pltpu.VMEM((WIN,), jnp.float32)
                    for c in range(G)})


@functools.partial(
    pl.kernel,
    out_type=jax.ShapeDtypeStruct((B, C, HW), jnp.float32),
    mesh=plsc.VectorSubcoreMesh(core_axis_name="c", subcore_axis_name="s",
                                num_cores=NC, num_subcores=NS),
    scratch_types=_SC_SCRATCH,
)
def _splat(in_hbm, idx_hbm, w_hbm, out_hbm, zbuf, fbuf, rnorm_v,
           sem_inA, sem_inB, sem_sc, sem_z, sem_out, **refs):
    sets = {}
    for p, sem in (("A", sem_inA), ("B", sem_inB)):
        sets[p] = (tuple(refs[f"idx{k}{p}"] for k in range(4)),
                   tuple(refs[f"wgt{k}{p}"] for k in range(4)),
                   refs[f"val{p}"], sem)
    upds = tuple(tuple(refs[f"upd{k}_{c}"] for c in range(G)) for k in range(4))
    planes = tuple(refs[f"plane{g}"] for g in range(G))
    obufs = tuple(refs[f"obuf{c}"] for c in range(G))
    cid = lax.axis_index("c")
    sid = lax.axis_index("s")

    def zfill(i):
        zbuf[pl.ds(i * L, L)] = jnp.zeros((L,), jnp.float32)
    pl.loop(0, ZCHUNK // L)(zfill)

    def zero_planes(ps):
        def zf(j):
            for p in ps:
                pltpu.async_copy(
                    zbuf, p.at[pl.ds(sid * SLICE + j * ZCHUNK, ZCHUNK)], sem_z)
        pl.loop(0, SLICE // ZCHUNK)(zf)

        def zw(j):
            for p in ps:
                pltpu.make_async_copy(
                    zbuf, p.at[pl.ds(sid * SLICE + j * ZCHUNK, ZCHUNK)],
                    sem_z).wait()
        pl.loop(0, SLICE // ZCHUNK)(zw)

    def stage_fire(w, nval, g, p):
        idxs, wgts, val_v, sem = sets[p]
        base = sid * SLICE + w * WIN
        for k in range(4):
            pltpu.async_copy(idx_hbm.at[cid, k, pl.ds(base, WIN)], idxs[k],
                             sem)
            pltpu.async_copy(w_hbm.at[cid, k, pl.ds(base, WIN)], wgts[k],
                             sem)
        for c in range(nval):
            pltpu.async_copy(in_hbm.at[cid, g * G + c, pl.ds(base, WIN)],
                             val_v.at[pl.ds(c * WIN, WIN)], sem)

    def stage_wait(w, nval, g, p):
        idxs, wgts, val_v, sem = sets[p]
        base = sid * SLICE + w * WIN
        for k in range(4):
            pltpu.make_async_copy(idx_hbm.at[cid, k, pl.ds(base, WIN)],
                                  idxs[k], sem).wait()
            pltpu.make_async_copy(w_hbm.at[cid, k, pl.ds(base, WIN)],
                                  wgts[k], sem).wait()
        for c in range(nval):
            pltpu.make_async_copy(in_hbm.at[cid, g * G + c, pl.ds(base, WIN)],
                                  val_v.at[pl.ds(c * WIN, WIN)], sem).wait()

    def paired_windows(nval, g, process):
        stage_fire(0, nval, g, "A")

        def pair(q):
            wa = 2 * q
            wb = 2 * q + 1
            wnext = jnp.minimum(wb + 1, NWIN - 1)
            stage_fire(wb, nval, g, "B")
            stage_wait(wa, nval, g, "A")
            process("A")
            stage_fire(wnext, nval, g, "A")
            stage_wait(wb, nval, g, "B")
            process("B")
        pl.loop(0, NWIN // 2)(pair)
        stage_wait(NWIN - 1, nval, g, "A")

    zero_planes(planes[:1])
    plsc.subcore_barrier()

    def nproc(p):
        idxs, wgts, _, _ = sets[p]
        for k in range(4):
            pltpu.async_copy(wgts[k], planes[0].at[idxs[k]], sem_sc, add=True)
        for k in range(4):
            pltpu.make_async_copy(wgts[k], planes[0].at[idxs[k]],
                                  sem_sc).wait()
    paired_windows(0, 0, nproc)
    plsc.subcore_barrier()

    def rext(j):
        base = sid * SLICE + j * WIN
        pltpu.sync_copy(planes[0].at[pl.ds(base, WIN)], fbuf)

        def rb(i):
            v16 = fbuf[pl.ds(i * L, L)]
            rnorm_v[pl.ds(j * WIN + i * L, L)] = 1.0 / jnp.where(
                v16 == 0.0, 1.0, v16)
        pl.loop(0, WIN // L)(rb)
    pl.loop(0, NWIN)(rext)

    def group(g):
        zero_planes(planes)
        plsc.subcore_barrier()

        def gproc(p):
            idxs, wgts, val_v, _ = sets[p]
            for k in range(4):
                def build(i, k=k):
                    s = pl.ds(i * L, L)
                    w16 = wgts[k][s]
                    for c in range(G):
                        upds[k][c][s] = w16 * val_v[pl.ds(c * WIN + i * L, L)]
                pl.loop(0, WIN // L)(build)
                for c in range(ABLATE_G):
                    pltpu.async_copy(upds[k][c], planes[c].at[idxs[k]],
                                     sem_sc, add=True)
            for k in range(4):
                for c in range(ABLATE_G):
                    pltpu.make_async_copy(upds[k][c], planes[c].at[idxs[k]],
                                          sem_sc).wait()
        paired_windows(G, g, gproc)
        plsc.subcore_barrier()

        def flush(j):
            base = sid * SLICE + j * WIN
            for c in range(G):
                pltpu.sync_copy(planes[c].at[pl.ds(base, WIN)], obufs[c])

                def fb(i, c=c):
                    s = pl.ds(i * L, L)
                    obufs[c][s] = obufs[c][s] * rnorm_v[pl.ds(j * WIN + i * L,
                                                              L)]
                pl.loop(0, WIN // L)(fb)
                pltpu.async_copy(obufs[c], out_hbm.at[cid, g * G + c,
                                                      pl.ds(base, WIN)],
                                 sem_out)
            for c in range(G):
                pltpu.make_async_copy(obufs[c],
                                      out_hbm.at[cid, g * G + c,
                                                 pl.ds(base, WIN)],
                                      sem_out).wait()
        pl.loop(0, NWIN)(flush)
    pl.loop(0, NGRP)(group)


def kernel(tenInput, tenFlow, tenMetric):
    idx4, w4 = _precompute(tenFlow, tenMetric)
    out = _splat(tenInput.reshape(B, C, HW),
                 idx4.reshape(B, 4, HW),
                 w4.reshape(B, 4, HW))
    return out.reshape(B, C, H, W)


# speedup vs baseline: 4.2282x; 1.2709x over previous
import functools

import jax
import jax.numpy as jnp
from jax import lax
from jax.experimental import pallas as pl
from jax.experimental.pallas import tpu as pltpu
from jax.experimental.pallas import tpu_sc as plsc

B, C, H, W = 2, 96, 512, 512
HW = H * W
NC, NS, L = 2, 16, 16
SLICE = HW // NS
WIN = 1024
NWIN = SLICE // WIN
G = 4
NGRP = C // G
ZCHUNK = 2048
ROWS_PER_BLK = 64


def _precompute_body(flow_ref, metric_ref, idx_ref, w_ref):
    r = pl.program_id(1) * ROWS_PER_BLK
    u = flow_ref[0, 0]
    v = flow_ref[0, 1]
    e = jnp.exp(metric_ref[0, 0])
    xi = lax.broadcasted_iota(jnp.int32, (ROWS_PER_BLK, W), 1)
    yi = lax.broadcasted_iota(jnp.int32, (ROWS_PER_BLK, W), 0) + r
    x = xi.astype(jnp.float32)
    y = yi.astype(jnp.float32)
    flt_x = x + u
    flt_y = y + v
    ix0f = jnp.floor(flt_x)
    iy0f = jnp.floor(flt_y)
    fx = flt_x - ix0f
    fy = flt_y - iy0f
    ix0 = ix0f.astype(jnp.int32)
    iy0 = iy0f.astype(jnp.int32)
    p_self = yi * W + xi
    for k, (ix, iy, wk) in enumerate((
            (ix0, iy0, (1.0 - fx) * (1.0 - fy)),
            (ix0 + 1, iy0, fx * (1.0 - fy)),
            (ix0, iy0 + 1, (1.0 - fx) * fy),
            (ix0 + 1, iy0 + 1, fx * fy),
    )):
        valid = (ix >= 0) & (ix < W) & (iy >= 0) & (iy < H)
        idx_ref[0, k] = jnp.where(valid, iy * W + ix, p_self)
        w_ref[0, k] = jnp.where(valid, wk * e, 0.0)


def _precompute(tenFlow, tenMetric):
    grid = (B, H // ROWS_PER_BLK)
    return pl.pallas_call(
        _precompute_body,
        grid=grid,
        in_specs=[
            pl.BlockSpec((1, 2, ROWS_PER_BLK, W), lambda b, i: (b, 0, i, 0)),
            pl.BlockSpec((1, 1, ROWS_PER_BLK, W), lambda b, i: (b, 0, i, 0)),
        ],
        out_specs=[
            pl.BlockSpec((1, 4, ROWS_PER_BLK, W), lambda b, i: (b, 0, i, 0)),
            pl.BlockSpec((1, 4, ROWS_PER_BLK, W), lambda b, i: (b, 0, i, 0)),
        ],
        out_shape=[
            jax.ShapeDtypeStruct((B, 4, H, W), jnp.int32),
            jax.ShapeDtypeStruct((B, 4, H, W), jnp.float32),
        ],
    )(tenFlow, tenMetric)


_SC_SCRATCH = dict(
    zbuf=pltpu.VMEM((ZCHUNK,), jnp.float32),
    fbuf=pltpu.VMEM((WIN,), jnp.float32),
    rnorm_v=pltpu.VMEM((SLICE,), jnp.float32),
    sem_inA=pltpu.SemaphoreType.DMA,
    sem_inB=pltpu.SemaphoreType.DMA,
    sem_sc=pltpu.SemaphoreType.DMA,
    sem_z=pltpu.SemaphoreType.DMA,
    sem_out=pltpu.SemaphoreType.DMA,
)
for _p in ("A", "B"):
    _SC_SCRATCH[f"val{_p}"] = pltpu.VMEM((G * WIN,), jnp.float32)
    _SC_SCRATCH.update({f"idx{k}{_p}": pltpu.VMEM((WIN,), jnp.int32)
                        for k in range(4)})
    _SC_SCRATCH.update({f"wgt{k}{_p}": pltpu.VMEM((WIN,), jnp.float32)
                        for k in range(4)})
_SC_SCRATCH.update({f"upd{k}_{c}": pltpu.VMEM((WIN,), jnp.float32)
                    for k in range(4) for c in range(G)})
_SC_SCRATCH.update({f"plane{g}": pltpu.VMEM_SHARED((HW,), jnp.float32)
                    for g in range(G)})
_SC_SCRATCH.update({f"obuf{c}": pltpu.VMEM((WIN,), jnp.float32)
                    for c in range(G)})


@functools.partial(
    pl.kernel,
    out_type=jax.ShapeDtypeStruct((B, C, HW), jnp.float32),
    mesh=plsc.VectorSubcoreMesh(core_axis_name="c", subcore_axis_name="s",
                                num_cores=NC, num_subcores=NS),
    scratch_types=_SC_SCRATCH,
)
def _splat(in_hbm, idx_hbm, w_hbm, out_hbm, zbuf, fbuf, rnorm_v,
           sem_inA, sem_inB, sem_sc, sem_z, sem_out, **refs):
    sets = {}
    for p, sem in (("A", sem_inA), ("B", sem_inB)):
        sets[p] = (tuple(refs[f"idx{k}{p}"] for k in range(4)),
                   tuple(refs[f"wgt{k}{p}"] for k in range(4)),
                   refs[f"val{p}"], sem)
    upds = tuple(tuple(refs[f"upd{k}_{c}"] for c in range(G)) for k in range(4))
    planes = tuple(refs[f"plane{g}"] for g in range(G))
    obufs = tuple(refs[f"obuf{c}"] for c in range(G))
    cid = lax.axis_index("c")
    sid = lax.axis_index("s")

    def zfill(i):
        zbuf[pl.ds(i * L, L)] = jnp.zeros((L,), jnp.float32)
    pl.loop(0, ZCHUNK // L)(zfill)

    def zero_planes(ps):
        def zf(j):
            for p in ps:
                pltpu.async_copy(
                    zbuf, p.at[pl.ds(sid * SLICE + j * ZCHUNK, ZCHUNK)], sem_z)
        pl.loop(0, SLICE // ZCHUNK)(zf)

        def zw(j):
            for p in ps:
                pltpu.make_async_copy(
                    zbuf, p.at[pl.ds(sid * SLICE + j * ZCHUNK, ZCHUNK)],
                    sem_z).wait()
        pl.loop(0, SLICE // ZCHUNK)(zw)

    def stage_fire(w, nval, g, p):
        idxs, wgts, val_v, sem = sets[p]
        base = sid * SLICE + w * WIN
        for k in range(4):
            pltpu.async_copy(idx_hbm.at[cid, k, pl.ds(base, WIN)], idxs[k],
                             sem)
            pltpu.async_copy(w_hbm.at[cid, k, pl.ds(base, WIN)], wgts[k],
                             sem)
        for c in range(nval):
            pltpu.async_copy(in_hbm.at[cid, g * G + c, pl.ds(base, WIN)],
                             val_v.at[pl.ds(c * WIN, WIN)], sem)

    def stage_wait(w, nval, g, p):
        idxs, wgts, val_v, sem = sets[p]
        base = sid * SLICE + w * WIN
        for k in range(4):
            pltpu.make_async_copy(idx_hbm.at[cid, k, pl.ds(base, WIN)],
                                  idxs[k], sem).wait()
            pltpu.make_async_copy(w_hbm.at[cid, k, pl.ds(base, WIN)],
                                  wgts[k], sem).wait()
        for c in range(nval):
            pltpu.make_async_copy(in_hbm.at[cid, g * G + c, pl.ds(base, WIN)],
                                  val_v.at[pl.ds(c * WIN, WIN)], sem).wait()

    def paired_windows(nval, g, process):
        stage_fire(0, nval, g, "A")

        def pair(q):
            wa = 2 * q
            wb = 2 * q + 1
            wnext = jnp.minimum(wb + 1, NWIN - 1)
            stage_fire(wb, nval, g, "B")
            stage_wait(wa, nval, g, "A")
            process("A")
            stage_fire(wnext, nval, g, "A")
            stage_wait(wb, nval, g, "B")
            process("B")
        pl.loop(0, NWIN // 2)(pair)
        stage_wait(NWIN - 1, nval, g, "A")

    zero_planes(planes[:1])
    plsc.subcore_barrier()

    def nproc(p):
        idxs, wgts, _, _ = sets[p]
        for k in range(4):
            pltpu.async_copy(wgts[k], planes[0].at[idxs[k]], sem_sc, add=True)
        for k in range(4):
            pltpu.make_async_copy(wgts[k], planes[0].at[idxs[k]],
                                  sem_sc).wait()
    paired_windows(0, 0, nproc)
    plsc.subcore_barrier()

    def rext(j):
        base = sid * SLICE + j * WIN
        pltpu.sync_copy(planes[0].at[pl.ds(base, WIN)], fbuf)

        @functools.partial(plsc.parallel_loop, 0, WIN // L, unroll=4)
        def rb(i):
            v16 = fbuf[pl.ds(i * L, L)]
            rnorm_v[pl.ds(j * WIN + i * L, L)] = 1.0 / jnp.where(
                v16 == 0.0, 1.0, v16)
    pl.loop(0, NWIN)(rext)

    def group(g):
        zero_planes(planes)
        plsc.subcore_barrier()

        def gproc(p):
            idxs, wgts, val_v, _ = sets[p]

            @functools.partial(plsc.parallel_loop, 0, WIN // L, unroll=4)
            def build(i):
                s = pl.ds(i * L, L)
                for k in range(4):
                    w16 = wgts[k][s]
                    for c in range(G):
                        upds[k][c][s] = w16 * val_v[pl.ds(c * WIN + i * L, L)]
            for k in range(4):
                for c in range(G):
                    pltpu.async_copy(upds[k][c], planes[c].at[idxs[k]],
                                     sem_sc, add=True)
            for k in range(4):
                for c in range(G):
                    pltpu.make_async_copy(upds[k][c], planes[c].at[idxs[k]],
                                          sem_sc).wait()
        paired_windows(G, g, gproc)
        plsc.subcore_barrier()

        def flush(j):
            base = sid * SLICE + j * WIN
            for c in range(G):
                pltpu.async_copy(planes[c].at[pl.ds(base, WIN)], obufs[c],
                                 sem_z)
            for c in range(G):
                pltpu.make_async_copy(planes[c].at[pl.ds(base, WIN)],
                                      obufs[c], sem_z).wait()

            @functools.partial(plsc.parallel_loop, 0, WIN // L, unroll=4)
            def fb(i):
                s = pl.ds(i * L, L)
                r16 = rnorm_v[pl.ds(j * WIN + i * L, L)]
                for c in range(G):
                    obufs[c][s] = obufs[c][s] * r16
            for c in range(G):
                pltpu.async_copy(obufs[c], out_hbm.at[cid, g * G + c,
                                                      pl.ds(base, WIN)],
                                 sem_out)
            for c in range(G):
                pltpu.make_async_copy(obufs[c],
                                      out_hbm.at[cid, g * G + c,
                                                 pl.ds(base, WIN)],
                                      sem_out).wait()
        pl.loop(0, NWIN)(flush)
    pl.loop(0, NGRP)(group)


def kernel(tenInput, tenFlow, tenMetric):
    idx4, w4 = _precompute(tenFlow, tenMetric)
    out = _splat(tenInput.reshape(B, C, HW),
                 idx4.reshape(B, 4, HW),
                 w4.reshape(B, 4, HW))
    return out.reshape(B, C, H, W)
